# baseline (speedup 1.0000x reference)
import functools

import jax
import jax.numpy as jnp
from jax import lax
from jax.experimental import pallas as pl
from jax.experimental.pallas import tpu as pltpu


def _round_up(n, m):
    return ((n + m - 1) // m) * m


def _count_embed_kernel(x_ref, emb_ref, out_ref, cnt_ref, *, inv_len,
                        num_full_tiles, vocab_tile, tail_rows):
    x = x_ref[...]
    L, _, TB = x.shape
    TV = vocab_tile
    xs = x.astype(jnp.int16)
    base_ids = lax.broadcasted_iota(jnp.int16, (L, TV, TB), 1)

    def tile_body(k, _):
        ids = base_ids + (k * TV).astype(jnp.int16)
        onehot = (xs == ids).astype(jnp.bfloat16)
        cnt_ref[pl.ds(pl.multiple_of(k * TV, TV), TV), :] = jnp.sum(
            onehot, axis=0)
        return _

    lax.fori_loop(0, num_full_tiles, tile_body, 0)

    base = num_full_tiles * TV
    ids = lax.broadcasted_iota(jnp.int16, (L, tail_rows, TB), 1) + jnp.int16(
        base)
    onehot = (xs == ids).astype(jnp.bfloat16)
    cnt_ref[base:base + tail_rows, :] = jnp.sum(onehot, axis=0)

    acc = lax.dot_general(cnt_ref[...], emb_ref[...],
                          (((0,), (0,)), ((), ())),
                          preferred_element_type=jnp.float32)
    out_ref[...] = (acc * jnp.float32(inv_len)).astype(out_ref.dtype)


@functools.partial(jax.jit, static_argnames=("batch_tile",))
def _count_morgan_embedding(x, emb_table, batch_tile=128):
    B, L = x.shape
    V, D = emb_table.shape

    TB = batch_tile
    TV = 128
    num_full_tiles = V // TV
    tail = V - num_full_tiles * TV
    tail_rows = _round_up(max(tail, 1), 16)
    V_pad = num_full_tiles * TV + tail_rows
    D_pad = _round_up(D, 128)

    x_t = jnp.transpose(x.astype(jnp.int32)).reshape(L, 1, B)
    emb_p = jnp.pad(emb_table.astype(jnp.bfloat16),
                    ((0, V_pad - V), (0, D_pad - D)))

    out = pl.pallas_call(
        functools.partial(_count_embed_kernel, inv_len=1.0 / L,
                          num_full_tiles=num_full_tiles, vocab_tile=TV,
                          tail_rows=tail_rows),
        out_shape=jax.ShapeDtypeStruct((B, D_pad), jnp.float32),
        grid_spec=pltpu.PrefetchScalarGridSpec(
            num_scalar_prefetch=0,
            grid=(B // TB,),
            in_specs=[
                pl.BlockSpec((L, 1, TB), lambda i: (0, 0, i)),
                pl.BlockSpec((V_pad, D_pad), lambda i: (0, 0)),
            ],
            out_specs=pl.BlockSpec((TB, D_pad), lambda i: (i, 0)),
            scratch_shapes=[pltpu.VMEM((V_pad, TB), jnp.bfloat16)],
        ),
        compiler_params=pltpu.CompilerParams(
            dimension_semantics=("parallel",)),
    )(x_t, emb_p)

    return out[:, :D].astype(emb_table.dtype)


def kernel(x, emb_table):
    return _count_morgan_embedding(x, emb_table)


# speedup vs baseline: 5.3570x; 5.3570x over previous
import functools

import jax
import jax.numpy as jnp
from jax import lax
from jax.experimental import pallas as pl
from jax.experimental.pallas import tpu as pltpu


def _round_up(n, m):
    return ((n + m - 1) // m) * m


def _count_embed_kernel(x_ref, emb_ref, out_ref, cnt_ref, *, inv_len,
                        num_full_tiles, vocab_tile, tail_rows):
    x = x_ref[...]
    L, _, TB = x.shape
    TV = vocab_tile
    xs = x.astype(jnp.int16)
    one = jnp.bfloat16(1)
    zero = jnp.bfloat16(0)
    base_ids = lax.broadcasted_iota(jnp.int16, (L, TV, TB), 1)

    def tile_body(k, carry):
        ids = base_ids + (k * TV).astype(jnp.int16)
        onehot = jnp.where(xs == ids, one, zero)
        cnt_ref[pl.ds(pl.multiple_of(k * TV, TV), TV), :] = jnp.sum(
            onehot, axis=0, dtype=jnp.bfloat16)
        return carry

    lax.fori_loop(0, num_full_tiles, tile_body, 0)

    base = num_full_tiles * TV
    ids = lax.broadcasted_iota(jnp.int16, (L, tail_rows, TB), 1) + jnp.int16(
        base)
    onehot = jnp.where(xs == ids, one, zero)
    cnt_ref[base:base + tail_rows, :] = jnp.sum(onehot, axis=0,
                                                dtype=jnp.bfloat16)

    acc = lax.dot_general(cnt_ref[...], emb_ref[...],
                          (((0,), (0,)), ((), ())),
                          preferred_element_type=jnp.float32)
    out_ref[...] = (acc * jnp.float32(inv_len)).astype(out_ref.dtype)


@functools.partial(jax.jit, static_argnames=("batch_tile",))
def _count_morgan_embedding(x, emb_table, batch_tile=128):
    B, L = x.shape
    V, D = emb_table.shape

    TB = batch_tile
    TV = 128
    num_full_tiles = V // TV
    tail = V - num_full_tiles * TV
    tail_rows = _round_up(max(tail, 1), 16)
    V_pad = num_full_tiles * TV + tail_rows
    D_pad = _round_up(D, 128)

    x_t = jnp.transpose(x.astype(jnp.int32)).reshape(L, 1, B)
    emb_p = jnp.pad(emb_table.astype(jnp.bfloat16),
                    ((0, V_pad - V), (0, D_pad - D)))

    out = pl.pallas_call(
        functools.partial(_count_embed_kernel, inv_len=1.0 / L,
                          num_full_tiles=num_full_tiles, vocab_tile=TV,
                          tail_rows=tail_rows),
        out_shape=jax.ShapeDtypeStruct((B, D_pad), jnp.float32),
        grid_spec=pltpu.PrefetchScalarGridSpec(
            num_scalar_prefetch=0,
            grid=(B // TB,),
            in_specs=[
                pl.BlockSpec((L, 1, TB), lambda i: (0, 0, i)),
                pl.BlockSpec((V_pad, D_pad), lambda i: (0, 0)),
            ],
            out_specs=pl.BlockSpec((TB, D_pad), lambda i: (i, 0)),
            scratch_shapes=[pltpu.VMEM((V_pad, TB), jnp.bfloat16)],
        ),
        compiler_params=pltpu.CompilerParams(
            dimension_semantics=("parallel",)),
    )(x_t, emb_p)

    return out[:, :D].astype(emb_table.dtype)


def kernel(x, emb_table):
    return _count_morgan_embedding(x, emb_table)


# speedup vs baseline: 5.6008x; 1.0455x over previous
import functools

import jax
import jax.numpy as jnp
from jax import lax
from jax.experimental import pallas as pl
from jax.experimental.pallas import tpu as pltpu


def _round_up(n, m):
    return ((n + m - 1) // m) * m


def _count_embed_kernel(x_ref, emb_ref, out_ref, cnt_ref, *, inv_len,
                        num_full_tiles, vocab_tile, tail_rows):
    x = x_ref[...]
    L, _, TB = x.shape
    TV = vocab_tile
    xs = x.astype(jnp.int16)
    one = jnp.bfloat16(1)
    zero = jnp.bfloat16(0)
    base_ids = lax.broadcasted_iota(jnp.int16, (L, TV, TB), 1)

    def tile_body(k, carry):
        ids = base_ids + (k * TV).astype(jnp.int16)
        onehot = jnp.where(xs == ids, one, zero)
        cnt_ref[pl.ds(pl.multiple_of(k * TV, TV), TV), :] = jnp.sum(
            onehot, axis=0, dtype=jnp.bfloat16)
        return carry

    lax.fori_loop(0, num_full_tiles, tile_body, 0)

    base = num_full_tiles * TV
    ids = lax.broadcasted_iota(jnp.int16, (L, tail_rows, TB), 1) + jnp.int16(
        base)
    onehot = jnp.where(xs == ids, one, zero)
    cnt_ref[base:base + tail_rows, :] = jnp.sum(onehot, axis=0,
                                                dtype=jnp.bfloat16)

    acc = lax.dot_general(cnt_ref[...], emb_ref[...],
                          (((0,), (0,)), ((), ())),
                          preferred_element_type=jnp.float32)
    out_ref[...] = (acc * jnp.float32(inv_len)).astype(out_ref.dtype)


@functools.partial(jax.jit, static_argnames=("batch_tile",))
def _count_morgan_embedding(x, emb_table, batch_tile=256):
    B, L = x.shape
    V, D = emb_table.shape

    TB = batch_tile
    TV = 128
    num_full_tiles = V // TV
    tail = V - num_full_tiles * TV
    tail_rows = _round_up(max(tail, 1), 16)
    V_pad = num_full_tiles * TV + tail_rows
    D_pad = _round_up(D, 128)

    x_t = jnp.transpose(x.astype(jnp.int32)).reshape(L, 1, B)
    emb_p = jnp.pad(emb_table.astype(jnp.bfloat16),
                    ((0, V_pad - V), (0, D_pad - D)))

    out = pl.pallas_call(
        functools.partial(_count_embed_kernel, inv_len=1.0 / L,
                          num_full_tiles=num_full_tiles, vocab_tile=TV,
                          tail_rows=tail_rows),
        out_shape=jax.ShapeDtypeStruct((B, D_pad), jnp.float32),
        grid_spec=pltpu.PrefetchScalarGridSpec(
            num_scalar_prefetch=0,
            grid=(B // TB,),
            in_specs=[
                pl.BlockSpec((L, 1, TB), lambda i: (0, 0, i)),
                pl.BlockSpec((V_pad, D_pad), lambda i: (0, 0)),
            ],
            out_specs=pl.BlockSpec((TB, D_pad), lambda i: (i, 0)),
            scratch_shapes=[pltpu.VMEM((V_pad, TB), jnp.bfloat16)],
        ),
        compiler_params=pltpu.CompilerParams(
            dimension_semantics=("parallel",)),
    )(x_t, emb_p)

    return out[:, :D].astype(emb_table.dtype)


def kernel(x, emb_table):
    return _count_morgan_embedding(x, emb_table)


# speedup vs baseline: 5.8131x; 1.0379x over previous
import functools

import jax
import jax.numpy as jnp
from jax import lax
from jax.experimental import pallas as pl
from jax.experimental.pallas import tpu as pltpu


def _round_up(n, m):
    return ((n + m - 1) // m) * m


def _count_embed_kernel(x_ref, emb_ref, out_ref, cnt_ref, *, inv_len,
                        num_full_tiles, vocab_tile, tail_rows):
    x = x_ref[...]
    L, _, TB = x.shape
    TV = vocab_tile
    xs = x.astype(jnp.int16)
    one = jnp.bfloat16(1)
    zero = jnp.bfloat16(0)
    base_ids = lax.broadcasted_iota(jnp.int16, (L, TV, TB), 1)

    def tile_body(k, carry):
        ids = base_ids + (k * TV).astype(jnp.int16)
        onehot = jnp.where(xs == ids, one, zero)
        cnt_ref[pl.ds(pl.multiple_of(k * TV, TV), TV), :] = jnp.sum(
            onehot, axis=0, dtype=jnp.bfloat16)
        return carry

    lax.fori_loop(0, num_full_tiles, tile_body, 0)

    base = num_full_tiles * TV
    ids = lax.broadcasted_iota(jnp.int16, (L, tail_rows, TB), 1) + jnp.int16(
        base)
    onehot = jnp.where(xs == ids, one, zero)
    cnt_ref[base:base + tail_rows, :] = jnp.sum(onehot, axis=0,
                                                dtype=jnp.bfloat16)

    acc = lax.dot_general(cnt_ref[...], emb_ref[...],
                          (((0,), (0,)), ((), ())),
                          preferred_element_type=jnp.float32)
    out_ref[...] = (acc * jnp.float32(inv_len)).astype(out_ref.dtype)


@functools.partial(jax.jit, static_argnames=("batch_tile",))
def _count_morgan_embedding(x, emb_table, batch_tile=256):
    B, L = x.shape
    V, D = emb_table.shape

    TB = batch_tile
    TV = 256
    num_full_tiles = V // TV
    tail = V - num_full_tiles * TV
    tail_rows = _round_up(max(tail, 1), 16)
    V_pad = num_full_tiles * TV + tail_rows
    D_pad = _round_up(D, 128)

    x_t = jnp.transpose(x.astype(jnp.int32)).reshape(L, 1, B)
    emb_p = jnp.pad(emb_table.astype(jnp.bfloat16),
                    ((0, V_pad - V), (0, D_pad - D)))

    out = pl.pallas_call(
        functools.partial(_count_embed_kernel, inv_len=1.0 / L,
                          num_full_tiles=num_full_tiles, vocab_tile=TV,
                          tail_rows=tail_rows),
        out_shape=jax.ShapeDtypeStruct((B, D_pad), jnp.float32),
        grid_spec=pltpu.PrefetchScalarGridSpec(
            num_scalar_prefetch=0,
            grid=(B // TB,),
            in_specs=[
                pl.BlockSpec((L, 1, TB), lambda i: (0, 0, i)),
                pl.BlockSpec((V_pad, D_pad), lambda i: (0, 0)),
            ],
            out_specs=pl.BlockSpec((TB, D_pad), lambda i: (i, 0)),
            scratch_shapes=[pltpu.VMEM((V_pad, TB), jnp.bfloat16)],
        ),
        compiler_params=pltpu.CompilerParams(
            dimension_semantics=("parallel",)),
    )(x_t, emb_p)

    return out[:, :D].astype(emb_table.dtype)


def kernel(x, emb_table):
    return _count_morgan_embedding(x, emb_table)


# speedup vs baseline: 5.8151x; 1.0003x over previous
import functools

import jax
import jax.numpy as jnp
from jax import lax
from jax.experimental import pallas as pl
from jax.experimental.pallas import tpu as pltpu


def _round_up(n, m):
    return ((n + m - 1) // m) * m


def _count_embed_kernel(x_ref, emb_ref, out_ref, cnt_ref, *, inv_len,
                        num_full_tiles, vocab_tile, tail_rows):
    x = x_ref[...]
    L, _, TB = x.shape
    TV = vocab_tile
    xs = x.astype(jnp.int16)
    one = jnp.bfloat16(1)
    zero = jnp.bfloat16(0)
    base_ids = lax.broadcasted_iota(jnp.int16, (1, TV, TB), 1)

    def tile_body(k, carry):
        ids = base_ids + (k * TV).astype(jnp.int16)
        onehot = jnp.where(xs == ids, one, zero)
        cnt_ref[pl.ds(pl.multiple_of(k * TV, TV), TV), :] = jnp.sum(
            onehot, axis=0, dtype=jnp.bfloat16)
        return carry

    lax.fori_loop(0, num_full_tiles, tile_body, 0)

    base = num_full_tiles * TV
    ids = lax.broadcasted_iota(jnp.int16, (L, tail_rows, TB), 1) + jnp.int16(
        base)
    onehot = jnp.where(xs == ids, one, zero)
    cnt_ref[base:base + tail_rows, :] = jnp.sum(onehot, axis=0,
                                                dtype=jnp.bfloat16)

    acc = lax.dot_general(cnt_ref[...], emb_ref[...],
                          (((0,), (0,)), ((), ())),
                          preferred_element_type=jnp.float32)
    out_ref[...] = (acc * jnp.float32(inv_len)).astype(out_ref.dtype)


@functools.partial(jax.jit, static_argnames=("batch_tile",))
def _count_morgan_embedding(x, emb_table, batch_tile=256):
    B, L = x.shape
    V, D = emb_table.shape

    TB = batch_tile
    TV = 256
    num_full_tiles = V // TV
    tail = V - num_full_tiles * TV
    tail_rows = _round_up(max(tail, 1), 16)
    V_pad = num_full_tiles * TV + tail_rows
    D_pad = _round_up(D, 128)

    x_t = jnp.transpose(x.astype(jnp.int32)).reshape(L, 1, B)
    emb_p = jnp.pad(emb_table.astype(jnp.bfloat16),
                    ((0, V_pad - V), (0, D_pad - D)))

    out = pl.pallas_call(
        functools.partial(_count_embed_kernel, inv_len=1.0 / L,
                          num_full_tiles=num_full_tiles, vocab_tile=TV,
                          tail_rows=tail_rows),
        out_shape=jax.ShapeDtypeStruct((B, D_pad), jnp.float32),
        grid_spec=pltpu.PrefetchScalarGridSpec(
            num_scalar_prefetch=0,
            grid=(B // TB,),
            in_specs=[
                pl.BlockSpec((L, 1, TB), lambda i: (0, 0, i)),
                pl.BlockSpec((V_pad, D_pad), lambda i: (0, 0)),
            ],
            out_specs=pl.BlockSpec((TB, D_pad), lambda i: (i, 0)),
            scratch_shapes=[pltpu.VMEM((V_pad, TB), jnp.bfloat16)],
        ),
        compiler_params=pltpu.CompilerParams(
            dimension_semantics=("parallel",)),
    )(x_t, emb_p)

    return out[:, :D].astype(emb_table.dtype)


def kernel(x, emb_table):
    return _count_morgan_embedding(x, emb_table)


# speedup vs baseline: 6.0092x; 1.0334x over previous
import functools

import jax
import jax.numpy as jnp
from jax import lax
from jax.experimental import pallas as pl
from jax.experimental.pallas import tpu as pltpu


def _round_up(n, m):
    return ((n + m - 1) // m) * m


def _count_embed_kernel(x_ref, emb_ref, out_ref, cnt_ref, *, inv_len,
                        num_full_tiles, vocab_tile, tail_rows):
    x = x_ref[...]
    L, _, TB = x.shape
    TV = vocab_tile
    xs = x.astype(jnp.int16)
    one = jnp.bfloat16(1)
    zero = jnp.bfloat16(0)
    base_ids = lax.broadcasted_iota(jnp.int16, (1, TV, TB), 1)

    def tile_body(k2, carry):
        for j in range(2):
            k = k2 * 2 + j
            ids = base_ids + (k * TV).astype(jnp.int16)
            onehot = jnp.where(xs == ids, one, zero)
            cnt_ref[pl.ds(pl.multiple_of(k * TV, TV), TV), :] = jnp.sum(
                onehot, axis=0, dtype=jnp.bfloat16)
        return carry

    lax.fori_loop(0, num_full_tiles // 2, tile_body, 0)

    base = num_full_tiles * TV
    ids = lax.broadcasted_iota(jnp.int16, (L, tail_rows, TB), 1) + jnp.int16(
        base)
    onehot = jnp.where(xs == ids, one, zero)
    cnt_ref[base:base + tail_rows, :] = jnp.sum(onehot, axis=0,
                                                dtype=jnp.bfloat16)

    acc = lax.dot_general(cnt_ref[...], emb_ref[...],
                          (((0,), (0,)), ((), ())),
                          preferred_element_type=jnp.float32)
    out_ref[...] = (acc * jnp.float32(inv_len)).astype(out_ref.dtype)


@functools.partial(jax.jit, static_argnames=("batch_tile",))
def _count_morgan_embedding(x, emb_table, batch_tile=256):
    B, L = x.shape
    V, D = emb_table.shape

    TB = batch_tile
    TV = 256
    num_full_tiles = V // TV
    tail = V - num_full_tiles * TV
    tail_rows = _round_up(max(tail, 1), 16)
    V_pad = num_full_tiles * TV + tail_rows
    D_pad = _round_up(D, 128)

    x_t = jnp.transpose(x.astype(jnp.int32)).reshape(L, 1, B)
    emb_p = jnp.pad(emb_table.astype(jnp.bfloat16),
                    ((0, V_pad - V), (0, D_pad - D)))

    out = pl.pallas_call(
        functools.partial(_count_embed_kernel, inv_len=1.0 / L,
                          num_full_tiles=num_full_tiles, vocab_tile=TV,
                          tail_rows=tail_rows),
        out_shape=jax.ShapeDtypeStruct((B, D_pad), jnp.float32),
        grid_spec=pltpu.PrefetchScalarGridSpec(
            num_scalar_prefetch=0,
            grid=(B // TB,),
            in_specs=[
                pl.BlockSpec((L, 1, TB), lambda i: (0, 0, i)),
                pl.BlockSpec((V_pad, D_pad), lambda i: (0, 0)),
            ],
            out_specs=pl.BlockSpec((TB, D_pad), lambda i: (i, 0)),
            scratch_shapes=[pltpu.VMEM((V_pad, TB), jnp.bfloat16)],
        ),
        compiler_params=pltpu.CompilerParams(
            dimension_semantics=("parallel",)),
    )(x_t, emb_p)

    return out[:, :D].astype(emb_table.dtype)


def kernel(x, emb_table):
    return _count_morgan_embedding(x, emb_table)


# speedup vs baseline: 6.1066x; 1.0162x over previous
import functools

import jax
import jax.numpy as jnp
from jax import lax
from jax.experimental import pallas as pl
from jax.experimental.pallas import tpu as pltpu


def _round_up(n, m):
    return ((n + m - 1) // m) * m


def _count_embed_kernel(x_ref, emb_ref, out_ref, cnt_ref, *, inv_len,
                        num_full_tiles, vocab_tile, tail_rows):
    x = x_ref[...]
    L, _, TB = x.shape
    TV = vocab_tile
    xs = x.astype(jnp.int16)
    one = jnp.bfloat16(1)
    zero = jnp.bfloat16(0)
    base_ids = lax.broadcasted_iota(jnp.int16, (1, TV, TB), 1)

    def tile_body(k2, carry):
        for j in range(4):
            k = k2 * 4 + j
            ids = base_ids + (k * TV).astype(jnp.int16)
            onehot = jnp.where(xs == ids, one, zero)
            cnt_ref[pl.ds(pl.multiple_of(k * TV, TV), TV), :] = jnp.sum(
                onehot, axis=0, dtype=jnp.bfloat16)
        return carry

    lax.fori_loop(0, num_full_tiles // 4, tile_body, 0)

    base = num_full_tiles * TV
    ids = lax.broadcasted_iota(jnp.int16, (L, tail_rows, TB), 1) + jnp.int16(
        base)
    onehot = jnp.where(xs == ids, one, zero)
    cnt_ref[base:base + tail_rows, :] = jnp.sum(onehot, axis=0,
                                                dtype=jnp.bfloat16)

    acc = lax.dot_general(cnt_ref[...], emb_ref[...],
                          (((0,), (0,)), ((), ())),
                          preferred_element_type=jnp.float32)
    out_ref[...] = (acc * jnp.float32(inv_len)).astype(out_ref.dtype)


@functools.partial(jax.jit, static_argnames=("batch_tile",))
def _count_morgan_embedding(x, emb_table, batch_tile=256):
    B, L = x.shape
    V, D = emb_table.shape

    TB = batch_tile
    TV = 256
    num_full_tiles = V // TV
    tail = V - num_full_tiles * TV
    tail_rows = _round_up(max(tail, 1), 16)
    V_pad = num_full_tiles * TV + tail_rows
    D_pad = _round_up(D, 128)

    x_t = jnp.transpose(x.astype(jnp.int32)).reshape(L, 1, B)
    emb_p = jnp.pad(emb_table.astype(jnp.bfloat16),
                    ((0, V_pad - V), (0, D_pad - D)))

    out = pl.pallas_call(
        functools.partial(_count_embed_kernel, inv_len=1.0 / L,
                          num_full_tiles=num_full_tiles, vocab_tile=TV,
                          tail_rows=tail_rows),
        out_shape=jax.ShapeDtypeStruct((B, D_pad), jnp.float32),
        grid_spec=pltpu.PrefetchScalarGridSpec(
            num_scalar_prefetch=0,
            grid=(B // TB,),
            in_specs=[
                pl.BlockSpec((L, 1, TB), lambda i: (0, 0, i)),
                pl.BlockSpec((V_pad, D_pad), lambda i: (0, 0)),
            ],
            out_specs=pl.BlockSpec((TB, D_pad), lambda i: (i, 0)),
            scratch_shapes=[pltpu.VMEM((V_pad, TB), jnp.bfloat16)],
        ),
        compiler_params=pltpu.CompilerParams(
            dimension_semantics=("parallel",)),
    )(x_t, emb_p)

    return out[:, :D].astype(emb_table.dtype)


def kernel(x, emb_table):
    return _count_morgan_embedding(x, emb_table)


# speedup vs baseline: 6.1624x; 1.0091x over previous
import functools

import jax
import jax.numpy as jnp
from jax import lax
from jax.experimental import pallas as pl
from jax.experimental.pallas import tpu as pltpu


def _round_up(n, m):
    return ((n + m - 1) // m) * m


def _count_embed_kernel(x_ref, emb_ref, out_ref, cnt_ref, *, inv_len,
                        num_full_tiles, vocab_tile, tail_rows):
    x = x_ref[...]
    L, _, TB = x.shape
    TV = vocab_tile
    xs = x.astype(jnp.int16)
    one = jnp.bfloat16(1)
    zero = jnp.bfloat16(0)
    base_ids = lax.broadcasted_iota(jnp.int16, (1, TV, TB), 1)

    for k in range(num_full_tiles):
        ids = base_ids + jnp.int16(k * TV)
        onehot = jnp.where(xs == ids, one, zero)
        cnt_ref[k * TV:(k + 1) * TV, :] = jnp.sum(
            onehot, axis=0, dtype=jnp.bfloat16)

    base = num_full_tiles * TV
    ids = lax.broadcasted_iota(jnp.int16, (L, tail_rows, TB), 1) + jnp.int16(
        base)
    onehot = jnp.where(xs == ids, one, zero)
    cnt_ref[base:base + tail_rows, :] = jnp.sum(onehot, axis=0,
                                                dtype=jnp.bfloat16)

    acc = lax.dot_general(cnt_ref[...], emb_ref[...],
                          (((0,), (0,)), ((), ())),
                          preferred_element_type=jnp.float32)
    out_ref[...] = (acc * jnp.float32(inv_len)).astype(out_ref.dtype)


@functools.partial(jax.jit, static_argnames=("batch_tile",))
def _count_morgan_embedding(x, emb_table, batch_tile=256):
    B, L = x.shape
    V, D = emb_table.shape

    TB = batch_tile
    TV = 256
    num_full_tiles = V // TV
    tail = V - num_full_tiles * TV
    tail_rows = _round_up(max(tail, 1), 16)
    V_pad = num_full_tiles * TV + tail_rows
    D_pad = _round_up(D, 128)

    x_t = jnp.transpose(x.astype(jnp.int32)).reshape(L, 1, B)
    emb_p = jnp.pad(emb_table.astype(jnp.bfloat16),
                    ((0, V_pad - V), (0, D_pad - D)))

    out = pl.pallas_call(
        functools.partial(_count_embed_kernel, inv_len=1.0 / L,
                          num_full_tiles=num_full_tiles, vocab_tile=TV,
                          tail_rows=tail_rows),
        out_shape=jax.ShapeDtypeStruct((B, D_pad), jnp.float32),
        grid_spec=pltpu.PrefetchScalarGridSpec(
            num_scalar_prefetch=0,
            grid=(B // TB,),
            in_specs=[
                pl.BlockSpec((L, 1, TB), lambda i: (0, 0, i)),
                pl.BlockSpec((V_pad, D_pad), lambda i: (0, 0)),
            ],
            out_specs=pl.BlockSpec((TB, D_pad), lambda i: (i, 0)),
            scratch_shapes=[pltpu.VMEM((V_pad, TB), jnp.bfloat16)],
        ),
        compiler_params=pltpu.CompilerParams(
            dimension_semantics=("parallel",)),
    )(x_t, emb_p)

    return out[:, :D].astype(emb_table.dtype)


def kernel(x, emb_table):
    return _count_morgan_embedding(x, emb_table)
